# Initial kernel scaffold; baseline (speedup 1.0000x reference)
import jax
import jax.numpy as jnp
from jax.experimental import pallas as pl


def kernel(T, adj, W_heads, a_heads, W_out, a_out):
    raise NotImplementedError("write your pallas kernel here")


# speedup vs baseline: 3252.5715x; 3252.5715x over previous
import jax
import jax.numpy as jnp
from jax.experimental import pallas as pl
from jax.experimental.pallas import tpu as pltpu

_N = 2048
_NFEAT = 128
_NHID = 8
_NCLASS = 32
_NHEADS = 8
_ALPHA = 0.2
_TILE_R = 128
_NTILES = _N // _TILE_R


def _elu(x):
    return jnp.where(x > 0, x, jnp.exp(x) - 1.0)


def _layer1_kernel(adj_ref, T_ref, Wh_ref, ah_ref, out_ref, h_s, c1_s, c2_s):
    i = pl.program_id(0)

    @pl.when(i == 0)
    def _prep():
        Tm = T_ref[...]
        for hd in range(_NHEADS):
            h = jnp.dot(Tm, Wh_ref[hd], preferred_element_type=jnp.float32,
                        precision=jax.lax.Precision.HIGHEST)
            h_s[hd] = h
            a1 = ah_ref[hd, 0, :_NHID]
            a2 = ah_ref[hd, 0, _NHID:]
            u = jnp.sum(h * a1[None, :], axis=1)
            v = jnp.sum(h * a2[None, :], axis=1)
            c1_s[2 * hd, :] = -u
            c1_s[2 * hd + 1, :] = -v
            c2_s[2 * hd, :] = -_ALPHA * u
            c2_s[2 * hd + 1, :] = -_ALPHA * v

    r0 = i * _TILE_R
    adj_t = adj_ref[...]
    for hd in range(_NHEADS):
        nu1 = c1_s[2 * hd, pl.ds(r0, _TILE_R)]
        nv1 = c1_s[2 * hd + 1, :]
        nu2 = c2_s[2 * hd, pl.ds(r0, _TILE_R)]
        nv2 = c2_s[2 * hd + 1, :]
        arg = jnp.minimum(nu1[:, None] + nv1[None, :],
                          nu2[:, None] + nv2[None, :])
        e = jnp.exp(arg) * adj_t
        rowsum = jnp.sum(e, axis=1)
        hp = jnp.dot(e, h_s[hd], preferred_element_type=jnp.float32,
                     precision=jax.lax.Precision.HIGHEST)
        out_ref[:, hd * _NHID:(hd + 1) * _NHID] = _elu(hp / rowsum[:, None])


def _layer2_kernel(adj_ref, x_ref, Wo_ref, ao_ref, out_ref, h_s, c1_s, c2_s):
    i = pl.program_id(0)

    @pl.when(i == 0)
    def _prep():
        h = jnp.dot(x_ref[...], Wo_ref[...], preferred_element_type=jnp.float32,
                    precision=jax.lax.Precision.HIGHEST)
        h_s[...] = h
        a1 = ao_ref[0, :_NCLASS]
        a2 = ao_ref[0, _NCLASS:]
        u = jnp.sum(h * a1[None, :], axis=1)
        v = jnp.sum(h * a2[None, :], axis=1)
        c1_s[0, :] = -u
        c1_s[1, :] = -v
        c2_s[0, :] = -_ALPHA * u
        c2_s[1, :] = -_ALPHA * v

    r0 = i * _TILE_R
    adj_t = adj_ref[...]
    nu1 = c1_s[0, pl.ds(r0, _TILE_R)]
    nv1 = c1_s[1, :]
    nu2 = c2_s[0, pl.ds(r0, _TILE_R)]
    nv2 = c2_s[1, :]
    arg = jnp.minimum(nu1[:, None] + nv1[None, :],
                      nu2[:, None] + nv2[None, :])
    e = jnp.exp(arg) * adj_t
    rowsum = jnp.sum(e, axis=1)
    hp = jnp.dot(e, h_s[...], preferred_element_type=jnp.float32,
                 precision=jax.lax.Precision.HIGHEST)
    y = _elu(hp / rowsum[:, None])
    m = jnp.max(y, axis=1, keepdims=True)
    z = y - m
    lse = jnp.log(jnp.sum(jnp.exp(z), axis=1, keepdims=True))
    out_ref[...] = z - lse


def kernel(T, adj, W_heads, a_heads, W_out, a_out):
    x1 = pl.pallas_call(
        _layer1_kernel,
        grid=(_NTILES,),
        in_specs=[
            pl.BlockSpec((_TILE_R, _N), lambda i: (i, 0)),
            pl.BlockSpec((_N, _NFEAT), lambda i: (0, 0)),
            pl.BlockSpec((_NHEADS, _NFEAT, _NHID), lambda i: (0, 0, 0)),
            pl.BlockSpec((_NHEADS, 1, 2 * _NHID), lambda i: (0, 0, 0)),
        ],
        out_specs=pl.BlockSpec((_TILE_R, _NHEADS * _NHID), lambda i: (i, 0)),
        out_shape=jax.ShapeDtypeStruct((_N, _NHEADS * _NHID), jnp.float32),
        scratch_shapes=[
            pltpu.VMEM((_NHEADS, _N, _NHID), jnp.float32),
            pltpu.VMEM((2 * _NHEADS, _N), jnp.float32),
            pltpu.VMEM((2 * _NHEADS, _N), jnp.float32),
        ],
        compiler_params=pltpu.CompilerParams(
            dimension_semantics=("arbitrary",)),
    )(adj, T, W_heads, a_heads)

    out = pl.pallas_call(
        _layer2_kernel,
        grid=(_NTILES,),
        in_specs=[
            pl.BlockSpec((_TILE_R, _N), lambda i: (i, 0)),
            pl.BlockSpec((_N, _NHEADS * _NHID), lambda i: (0, 0)),
            pl.BlockSpec((_NHEADS * _NHID, _NCLASS), lambda i: (0, 0)),
            pl.BlockSpec((1, 2 * _NCLASS), lambda i: (0, 0)),
        ],
        out_specs=pl.BlockSpec((_TILE_R, _NCLASS), lambda i: (i, 0)),
        out_shape=jax.ShapeDtypeStruct((_N, _NCLASS), jnp.float32),
        scratch_shapes=[
            pltpu.VMEM((_N, _NCLASS), jnp.float32),
            pltpu.VMEM((2, _N), jnp.float32),
            pltpu.VMEM((2, _N), jnp.float32),
        ],
        compiler_params=pltpu.CompilerParams(
            dimension_semantics=("arbitrary",)),
    )(adj, x1, W_out, a_out)
    return out


# speedup vs baseline: 7452.7498x; 2.2913x over previous
import jax
import jax.numpy as jnp
from jax.experimental import pallas as pl
from jax.experimental.pallas import tpu as pltpu

_N = 2048
_NFEAT = 128
_NHID = 8
_NCLASS = 32
_NHEADS = 8
_ALPHA = 0.2
_TILE_R = 128
_NTILES = _N // _TILE_R


def _elu(x):
    return jnp.where(x > 0, x, jnp.exp(x) - 1.0)


def _layer1_kernel(adj_ref, T_ref, Wh_ref, ah_ref, out_ref, h_s, c1_s, c2_s):
    i = pl.program_id(0)

    @pl.when(i == 0)
    def _prep():
        Tm = T_ref[...]
        for hd in range(_NHEADS):
            h = jnp.dot(Tm, Wh_ref[hd], preferred_element_type=jnp.float32,
                        precision=jax.lax.Precision.HIGHEST)
            h_s[hd] = h
            a1 = ah_ref[hd, 0, :_NHID]
            a2 = ah_ref[hd, 0, _NHID:]
            u = jnp.sum(h * a1[None, :], axis=1)
            v = jnp.sum(h * a2[None, :], axis=1)
            c1_s[2 * hd, :] = -u
            c1_s[2 * hd + 1, :] = -v
            c2_s[2 * hd, :] = -_ALPHA * u
            c2_s[2 * hd + 1, :] = -_ALPHA * v

    r0 = i * _TILE_R
    adj_t = adj_ref[...]
    for hd in range(_NHEADS):
        nu1 = c1_s[2 * hd, pl.ds(r0, _TILE_R)]
        nv1 = c1_s[2 * hd + 1, :]
        nu2 = c2_s[2 * hd, pl.ds(r0, _TILE_R)]
        nv2 = c2_s[2 * hd + 1, :]
        arg = jnp.minimum(nu1[:, None] + nv1[None, :],
                          nu2[:, None] + nv2[None, :])
        e = jnp.exp(arg) * adj_t
        rowsum = jnp.sum(e, axis=1)
        hp = jnp.dot(e, h_s[hd], preferred_element_type=jnp.float32)
        out_ref[:, hd * _NHID:(hd + 1) * _NHID] = _elu(hp / rowsum[:, None])


def _layer2_kernel(adj_ref, x_ref, Wo_ref, ao_ref, out_ref, h_s, c1_s, c2_s):
    i = pl.program_id(0)

    @pl.when(i == 0)
    def _prep():
        h = jnp.dot(x_ref[...], Wo_ref[...], preferred_element_type=jnp.float32,
                    precision=jax.lax.Precision.HIGHEST)
        h_s[...] = h
        a1 = ao_ref[0, :_NCLASS]
        a2 = ao_ref[0, _NCLASS:]
        u = jnp.sum(h * a1[None, :], axis=1)
        v = jnp.sum(h * a2[None, :], axis=1)
        c1_s[0, :] = -u
        c1_s[1, :] = -v
        c2_s[0, :] = -_ALPHA * u
        c2_s[1, :] = -_ALPHA * v

    r0 = i * _TILE_R
    adj_t = adj_ref[...]
    nu1 = c1_s[0, pl.ds(r0, _TILE_R)]
    nv1 = c1_s[1, :]
    nu2 = c2_s[0, pl.ds(r0, _TILE_R)]
    nv2 = c2_s[1, :]
    arg = jnp.minimum(nu1[:, None] + nv1[None, :],
                      nu2[:, None] + nv2[None, :])
    e = jnp.exp(arg) * adj_t
    rowsum = jnp.sum(e, axis=1)
    hp = jnp.dot(e, h_s[...], preferred_element_type=jnp.float32)
    y = _elu(hp / rowsum[:, None])
    m = jnp.max(y, axis=1, keepdims=True)
    z = y - m
    lse = jnp.log(jnp.sum(jnp.exp(z), axis=1, keepdims=True))
    out_ref[...] = z - lse


def kernel(T, adj, W_heads, a_heads, W_out, a_out):
    x1 = pl.pallas_call(
        _layer1_kernel,
        grid=(_NTILES,),
        in_specs=[
            pl.BlockSpec((_TILE_R, _N), lambda i: (i, 0)),
            pl.BlockSpec((_N, _NFEAT), lambda i: (0, 0)),
            pl.BlockSpec((_NHEADS, _NFEAT, _NHID), lambda i: (0, 0, 0)),
            pl.BlockSpec((_NHEADS, 1, 2 * _NHID), lambda i: (0, 0, 0)),
        ],
        out_specs=pl.BlockSpec((_TILE_R, _NHEADS * _NHID), lambda i: (i, 0)),
        out_shape=jax.ShapeDtypeStruct((_N, _NHEADS * _NHID), jnp.float32),
        scratch_shapes=[
            pltpu.VMEM((_NHEADS, _N, _NHID), jnp.float32),
            pltpu.VMEM((2 * _NHEADS, _N), jnp.float32),
            pltpu.VMEM((2 * _NHEADS, _N), jnp.float32),
        ],
        compiler_params=pltpu.CompilerParams(
            dimension_semantics=("arbitrary",)),
    )(adj, T, W_heads, a_heads)

    out = pl.pallas_call(
        _layer2_kernel,
        grid=(_NTILES,),
        in_specs=[
            pl.BlockSpec((_TILE_R, _N), lambda i: (i, 0)),
            pl.BlockSpec((_N, _NHEADS * _NHID), lambda i: (0, 0)),
            pl.BlockSpec((_NHEADS * _NHID, _NCLASS), lambda i: (0, 0)),
            pl.BlockSpec((1, 2 * _NCLASS), lambda i: (0, 0)),
        ],
        out_specs=pl.BlockSpec((_TILE_R, _NCLASS), lambda i: (i, 0)),
        out_shape=jax.ShapeDtypeStruct((_N, _NCLASS), jnp.float32),
        scratch_shapes=[
            pltpu.VMEM((_N, _NCLASS), jnp.float32),
            pltpu.VMEM((2, _N), jnp.float32),
            pltpu.VMEM((2, _N), jnp.float32),
        ],
        compiler_params=pltpu.CompilerParams(
            dimension_semantics=("arbitrary",)),
    )(adj, x1, W_out, a_out)
    return out


# speedup vs baseline: 7647.3678x; 1.0261x over previous
import jax
import jax.numpy as jnp
from jax.experimental import pallas as pl
from jax.experimental.pallas import tpu as pltpu

_N = 2048
_NFEAT = 128
_NHID = 8
_NCLASS = 32
_NHEADS = 8
_ALPHA = 0.2
_TILE_R = 128
_NTILES = _N // _TILE_R


def _elu(x):
    return jnp.where(x > 0, x, jnp.exp(x) - 1.0)


def _layer1_kernel(adj_ref, T_ref, Wh_ref, ah_ref, out_ref, h_s, c1_s, c2_s):
    i = pl.program_id(0)

    @pl.when(i == 0)
    def _prep():
        Tm = T_ref[...]
        for hd in range(_NHEADS):
            h = jnp.dot(Tm, Wh_ref[hd], preferred_element_type=jnp.float32,
                        precision=jax.lax.Precision.HIGHEST)
            h_s[hd] = h.astype(jnp.bfloat16)
            a1 = ah_ref[hd, 0, :_NHID]
            a2 = ah_ref[hd, 0, _NHID:]
            u = jnp.sum(h * a1[None, :], axis=1)
            v = jnp.sum(h * a2[None, :], axis=1)
            c1_s[2 * hd, :] = -u
            c1_s[2 * hd + 1, :] = -v
            c2_s[2 * hd, :] = -_ALPHA * u
            c2_s[2 * hd + 1, :] = -_ALPHA * v

    r0 = i * _TILE_R
    adj_t = adj_ref[...]
    for hd in range(_NHEADS):
        nu1 = c1_s[2 * hd, pl.ds(r0, _TILE_R)]
        nv1 = c1_s[2 * hd + 1, :]
        nu2 = c2_s[2 * hd, pl.ds(r0, _TILE_R)]
        nv2 = c2_s[2 * hd + 1, :]
        arg = jnp.minimum(nu1[:, None] + nv1[None, :],
                          nu2[:, None] + nv2[None, :])
        e = jnp.exp(arg) * adj_t
        rowsum = jnp.sum(e, axis=1)
        hp = jnp.dot(e.astype(jnp.bfloat16), h_s[hd],
                     preferred_element_type=jnp.float32)
        out_ref[:, hd * _NHID:(hd + 1) * _NHID] = _elu(hp / rowsum[:, None])


def _layer2_kernel(adj_ref, x_ref, Wo_ref, ao_ref, out_ref, h_s, c1_s, c2_s):
    i = pl.program_id(0)

    @pl.when(i == 0)
    def _prep():
        h = jnp.dot(x_ref[...], Wo_ref[...], preferred_element_type=jnp.float32,
                    precision=jax.lax.Precision.HIGHEST)
        h_s[...] = h.astype(jnp.bfloat16)
        a1 = ao_ref[0, :_NCLASS]
        a2 = ao_ref[0, _NCLASS:]
        u = jnp.sum(h * a1[None, :], axis=1)
        v = jnp.sum(h * a2[None, :], axis=1)
        c1_s[0, :] = -u
        c1_s[1, :] = -v
        c2_s[0, :] = -_ALPHA * u
        c2_s[1, :] = -_ALPHA * v

    r0 = i * _TILE_R
    adj_t = adj_ref[...]
    nu1 = c1_s[0, pl.ds(r0, _TILE_R)]
    nv1 = c1_s[1, :]
    nu2 = c2_s[0, pl.ds(r0, _TILE_R)]
    nv2 = c2_s[1, :]
    arg = jnp.minimum(nu1[:, None] + nv1[None, :],
                      nu2[:, None] + nv2[None, :])
    e = jnp.exp(arg) * adj_t
    rowsum = jnp.sum(e, axis=1)
    hp = jnp.dot(e.astype(jnp.bfloat16), h_s[...],
                 preferred_element_type=jnp.float32)
    y = _elu(hp / rowsum[:, None])
    m = jnp.max(y, axis=1, keepdims=True)
    z = y - m
    lse = jnp.log(jnp.sum(jnp.exp(z), axis=1, keepdims=True))
    out_ref[...] = z - lse


def kernel(T, adj, W_heads, a_heads, W_out, a_out):
    x1 = pl.pallas_call(
        _layer1_kernel,
        grid=(_NTILES,),
        in_specs=[
            pl.BlockSpec((_TILE_R, _N), lambda i: (i, 0)),
            pl.BlockSpec((_N, _NFEAT), lambda i: (0, 0)),
            pl.BlockSpec((_NHEADS, _NFEAT, _NHID), lambda i: (0, 0, 0)),
            pl.BlockSpec((_NHEADS, 1, 2 * _NHID), lambda i: (0, 0, 0)),
        ],
        out_specs=pl.BlockSpec((_TILE_R, _NHEADS * _NHID), lambda i: (i, 0)),
        out_shape=jax.ShapeDtypeStruct((_N, _NHEADS * _NHID), jnp.float32),
        scratch_shapes=[
            pltpu.VMEM((_NHEADS, _N, _NHID), jnp.bfloat16),
            pltpu.VMEM((2 * _NHEADS, _N), jnp.float32),
            pltpu.VMEM((2 * _NHEADS, _N), jnp.float32),
        ],
        compiler_params=pltpu.CompilerParams(
            dimension_semantics=("arbitrary",)),
    )(adj, T, W_heads, a_heads)

    out = pl.pallas_call(
        _layer2_kernel,
        grid=(_NTILES,),
        in_specs=[
            pl.BlockSpec((_TILE_R, _N), lambda i: (i, 0)),
            pl.BlockSpec((_N, _NHEADS * _NHID), lambda i: (0, 0)),
            pl.BlockSpec((_NHEADS * _NHID, _NCLASS), lambda i: (0, 0)),
            pl.BlockSpec((1, 2 * _NCLASS), lambda i: (0, 0)),
        ],
        out_specs=pl.BlockSpec((_TILE_R, _NCLASS), lambda i: (i, 0)),
        out_shape=jax.ShapeDtypeStruct((_N, _NCLASS), jnp.float32),
        scratch_shapes=[
            pltpu.VMEM((_N, _NCLASS), jnp.bfloat16),
            pltpu.VMEM((2, _N), jnp.float32),
            pltpu.VMEM((2, _N), jnp.float32),
        ],
        compiler_params=pltpu.CompilerParams(
            dimension_semantics=("arbitrary",)),
    )(adj, x1, W_out, a_out)
    return out


# speedup vs baseline: 11589.8537x; 1.5155x over previous
import jax
import jax.numpy as jnp
from jax.experimental import pallas as pl
from jax.experimental.pallas import tpu as pltpu

_N = 2048
_NFEAT = 128
_NHID = 8
_NCLASS = 32
_NHEADS = 8
_ALPHA = 0.2
_TILE_R = 128
_NTILES = _N // _TILE_R


def _elu(x):
    return jnp.where(x > 0, x, jnp.exp(x) - 1.0)


def _layer1_kernel(adj_ref, T_ref, Wh_ref, ah_ref, out_ref,
                   h_s, cu1_s, cu2_s, cv1_s, cv2_s):
    i = pl.program_id(0)

    @pl.when(i == 0)
    def _prep():
        Tm = T_ref[...]
        ones = jnp.ones((_N, 1), dtype=jnp.bfloat16)
        for hd in range(_NHEADS):
            h = jnp.dot(Tm, Wh_ref[hd], preferred_element_type=jnp.float32,
                        precision=jax.lax.Precision.HIGHEST)
            h_s[hd] = jnp.concatenate([h.astype(jnp.bfloat16), ones], axis=1)
            a1 = ah_ref[hd, 0, :_NHID]
            a2 = ah_ref[hd, 0, _NHID:]
            u = jnp.sum(h * a1[None, :], axis=1, keepdims=True)
            v = jnp.sum(h * a2[None, :], axis=1, keepdims=True)
            vr = v.T
            cu1_s[:, hd:hd + 1] = -u
            cu2_s[:, hd:hd + 1] = -_ALPHA * u
            cv1_s[hd:hd + 1, :] = -vr
            cv2_s[hd:hd + 1, :] = -_ALPHA * vr

    r0 = i * _TILE_R
    adj_t = adj_ref[...]
    for hd in range(_NHEADS):
        nu1 = cu1_s[pl.ds(r0, _TILE_R), hd:hd + 1]
        nu2 = cu2_s[pl.ds(r0, _TILE_R), hd:hd + 1]
        nv1 = cv1_s[hd:hd + 1, :]
        nv2 = cv2_s[hd:hd + 1, :]
        arg = jnp.minimum(nu1 + nv1, nu2 + nv2)
        e = (jnp.exp(arg) * adj_t).astype(jnp.bfloat16)
        res = jnp.dot(e, h_s[hd], preferred_element_type=jnp.float32)
        hp = res[:, :_NHID]
        rowsum = res[:, _NHID:_NHID + 1]
        out_ref[:, hd * _NHID:(hd + 1) * _NHID] = _elu(hp / rowsum)


def _layer2_kernel(adj_ref, x_ref, Wo_ref, ao_ref, out_ref,
                   h_s, cu1_s, cu2_s, cv1_s, cv2_s):
    i = pl.program_id(0)

    @pl.when(i == 0)
    def _prep():
        h = jnp.dot(x_ref[...], Wo_ref[...], preferred_element_type=jnp.float32,
                    precision=jax.lax.Precision.HIGHEST)
        ones = jnp.ones((_N, 1), dtype=jnp.bfloat16)
        h_s[...] = jnp.concatenate([h.astype(jnp.bfloat16), ones], axis=1)
        a1 = ao_ref[0, :_NCLASS]
        a2 = ao_ref[0, _NCLASS:]
        u = jnp.sum(h * a1[None, :], axis=1, keepdims=True)
        v = jnp.sum(h * a2[None, :], axis=1, keepdims=True)
        vr = v.T
        cu1_s[...] = -u
        cu2_s[...] = -_ALPHA * u
        cv1_s[...] = -vr
        cv2_s[...] = -_ALPHA * vr

    r0 = i * _TILE_R
    adj_t = adj_ref[...]
    nu1 = cu1_s[pl.ds(r0, _TILE_R), :]
    nu2 = cu2_s[pl.ds(r0, _TILE_R), :]
    nv1 = cv1_s[...]
    nv2 = cv2_s[...]
    arg = jnp.minimum(nu1 + nv1, nu2 + nv2)
    e = (jnp.exp(arg) * adj_t).astype(jnp.bfloat16)
    res = jnp.dot(e, h_s[...], preferred_element_type=jnp.float32)
    hp = res[:, :_NCLASS]
    rowsum = res[:, _NCLASS:_NCLASS + 1]
    y = _elu(hp / rowsum)
    m = jnp.max(y, axis=1, keepdims=True)
    z = y - m
    lse = jnp.log(jnp.sum(jnp.exp(z), axis=1, keepdims=True))
    out_ref[...] = z - lse


def kernel(T, adj, W_heads, a_heads, W_out, a_out):
    x1 = pl.pallas_call(
        _layer1_kernel,
        grid=(_NTILES,),
        in_specs=[
            pl.BlockSpec((_TILE_R, _N), lambda i: (i, 0)),
            pl.BlockSpec((_N, _NFEAT), lambda i: (0, 0)),
            pl.BlockSpec((_NHEADS, _NFEAT, _NHID), lambda i: (0, 0, 0)),
            pl.BlockSpec((_NHEADS, 1, 2 * _NHID), lambda i: (0, 0, 0)),
        ],
        out_specs=pl.BlockSpec((_TILE_R, _NHEADS * _NHID), lambda i: (i, 0)),
        out_shape=jax.ShapeDtypeStruct((_N, _NHEADS * _NHID), jnp.float32),
        scratch_shapes=[
            pltpu.VMEM((_NHEADS, _N, _NHID + 1), jnp.bfloat16),
            pltpu.VMEM((_N, _NHEADS), jnp.float32),
            pltpu.VMEM((_N, _NHEADS), jnp.float32),
            pltpu.VMEM((_NHEADS, _N), jnp.float32),
            pltpu.VMEM((_NHEADS, _N), jnp.float32),
        ],
        compiler_params=pltpu.CompilerParams(
            dimension_semantics=("arbitrary",)),
    )(adj, T, W_heads, a_heads)

    out = pl.pallas_call(
        _layer2_kernel,
        grid=(_NTILES,),
        in_specs=[
            pl.BlockSpec((_TILE_R, _N), lambda i: (i, 0)),
            pl.BlockSpec((_N, _NHEADS * _NHID), lambda i: (0, 0)),
            pl.BlockSpec((_NHEADS * _NHID, _NCLASS), lambda i: (0, 0)),
            pl.BlockSpec((1, 2 * _NCLASS), lambda i: (0, 0)),
        ],
        out_specs=pl.BlockSpec((_TILE_R, _NCLASS), lambda i: (i, 0)),
        out_shape=jax.ShapeDtypeStruct((_N, _NCLASS), jnp.float32),
        scratch_shapes=[
            pltpu.VMEM((_N, _NCLASS + 1), jnp.bfloat16),
            pltpu.VMEM((_N, 1), jnp.float32),
            pltpu.VMEM((_N, 1), jnp.float32),
            pltpu.VMEM((1, _N), jnp.float32),
            pltpu.VMEM((1, _N), jnp.float32),
        ],
        compiler_params=pltpu.CompilerParams(
            dimension_semantics=("arbitrary",)),
    )(adj, x1, W_out, a_out)
    return out
